# Initial kernel scaffold; baseline (speedup 1.0000x reference)
import jax
import jax.numpy as jnp
from jax.experimental import pallas as pl


def kernel(src, table):
    raise NotImplementedError("write your pallas kernel here")


# speedup vs baseline: 4.9389x; 4.9389x over previous
import functools

import jax
import jax.numpy as jnp
from jax import lax
from jax.experimental import pallas as pl
from jax.experimental.pallas import tpu as pltpu
from jax.experimental.pallas import tpu_sc as plsc

NC = 2
NS = 16
NW = NC * NS
D = 32
G = 128
GPC = 16
C = G * GPC


@functools.partial(jax.jit, static_argnums=(2,))
def _emb_lookup(idx2d, table, tot):
    per_w = tot // NW
    n_chunks = per_w // C
    rows_per_w = per_w // G

    mesh = plsc.VectorSubcoreMesh(core_axis_name="c", subcore_axis_name="s")

    @functools.partial(
        pl.kernel,
        out_type=jax.ShapeDtypeStruct((tot, D), jnp.float32),
        mesh=mesh,
        scratch_types=[
            pltpu.VMEM((GPC, G), jnp.int32),
            pltpu.VMEM((C, D), jnp.float32),
            pltpu.SemaphoreType.DMA,
        ],
        compiler_params=pltpu.CompilerParams(use_tc_tiling_on_sc=False),
    )
    def k(idx_hbm, table_hbm, out_hbm, idx_v, rows_v, sem):
        wid = lax.axis_index("s") * NC + lax.axis_index("c")
        base_row = wid * rows_per_w

        @pl.loop(0, n_chunks)
        def chunk_loop(ci):
            row0 = base_row + ci * GPC
            pltpu.sync_copy(idx_hbm.at[pl.ds(row0, GPC)], idx_v)
            cps = [
                pltpu.async_copy(
                    table_hbm.at[idx_v.at[j]],
                    rows_v.at[pl.ds(j * G, G)],
                    sem,
                )
                for j in range(GPC)
            ]
            for cp in cps:
                cp.wait()
            pltpu.sync_copy(rows_v, out_hbm.at[pl.ds(row0 * G, C)])

    return k(idx2d, table)


def kernel(src, table):
    b, s = src.shape
    tot = b * s
    idx2d = src.reshape(tot // G, G).astype(jnp.int32)
    out = _emb_lookup(idx2d, table, tot)
    return out.reshape(b, s, D)


# speedup vs baseline: 4.9467x; 1.0016x over previous
import functools

import jax
import jax.numpy as jnp
from jax import lax
from jax.experimental import pallas as pl
from jax.experimental.pallas import tpu as pltpu
from jax.experimental.pallas import tpu_sc as plsc

NC = 2
NS = 16
NW = NC * NS
D = 32
G = 128
GPC = 8
C = G * GPC


@functools.partial(jax.jit, static_argnums=(2,))
def _emb_lookup(idx2d, table, tot):
    per_w = tot // NW
    n_chunks = per_w // C
    rows_per_w = per_w // G
    assert n_chunks % 2 == 0

    mesh = plsc.VectorSubcoreMesh(core_axis_name="c", subcore_axis_name="s")

    @functools.partial(
        pl.kernel,
        out_type=jax.ShapeDtypeStruct((tot, D), jnp.float32),
        mesh=mesh,
        scratch_types=[
            pltpu.VMEM((GPC, G), jnp.int32),
            pltpu.VMEM((GPC, G), jnp.int32),
            pltpu.VMEM((C, D), jnp.float32),
            pltpu.VMEM((C, D), jnp.float32),
            pltpu.SemaphoreType.DMA,
            pltpu.SemaphoreType.DMA,
        ],
        compiler_params=pltpu.CompilerParams(use_tc_tiling_on_sc=False),
    )
    def k(idx_hbm, table_hbm, out_hbm, idx_v0, idx_v1, rows_v0, rows_v1,
          gsem, wsem):
        idx_v = (idx_v0, idx_v1)
        rows_v = (rows_v0, rows_v1)
        wid = lax.axis_index("s") * NC + lax.axis_index("c")
        base_row = wid * rows_per_w

        def fire_gathers(ci, b):
            row0 = base_row + ci * GPC
            pltpu.sync_copy(idx_hbm.at[pl.ds(row0, GPC)], idx_v[b])
            for j in range(GPC):
                pltpu.async_copy(
                    table_hbm.at[idx_v[b].at[j]],
                    rows_v[b].at[pl.ds(j * G, G)],
                    gsem,
                )

        def wait_gathers(b):
            for j in range(GPC):
                pltpu.make_async_copy(
                    table_hbm.at[idx_v[b].at[j]],
                    rows_v[b].at[pl.ds(j * G, G)],
                    gsem,
                ).wait()

        def fire_write(ci, b):
            row0 = base_row + ci * GPC
            pltpu.async_copy(rows_v[b], out_hbm.at[pl.ds(row0 * G, C)], wsem)

        def wait_write(ci, b):
            row0 = base_row + ci * GPC
            pltpu.make_async_copy(
                rows_v[b], out_hbm.at[pl.ds(row0 * G, C)], wsem
            ).wait()

        fire_gathers(0, 0)

        @pl.loop(0, n_chunks, step=2)
        def chunk_loop(ci0):
            for b in range(2):
                ci = ci0 + b
                nb = 1 - b

                @pl.when(ci >= 1)
                def _():
                    wait_write(ci - 1, nb)

                @pl.when(ci + 1 < n_chunks)
                def _():
                    fire_gathers(ci + 1, nb)

                wait_gathers(b)
                fire_write(ci, b)

        wait_write(n_chunks - 1, (n_chunks - 1) % 2)

    return k(idx2d, table)


def kernel(src, table):
    b, s = src.shape
    tot = b * s
    idx2d = src.reshape(tot // G, G).astype(jnp.int32)
    out = _emb_lookup(idx2d, table, tot)
    return out.reshape(b, s, D)


# speedup vs baseline: 10.4996x; 2.1225x over previous
import functools

import jax
import jax.numpy as jnp
from jax import lax
from jax.experimental import pallas as pl
from jax.experimental.pallas import tpu as pltpu
from jax.experimental.pallas import tpu_sc as plsc

NC = 2
NS = 16
NW = NC * NS
D = 32
S = 200
G = 128
GPC = 8
C = G * GPC

BB = 128
PPB = BB * S // 4


def _gather_kernel(tot):
    per_w = tot // NW
    n_chunks = per_w // C
    rows_per_w = per_w // G
    assert n_chunks % 2 == 0

    mesh = plsc.VectorSubcoreMesh(core_axis_name="c", subcore_axis_name="s")

    @functools.partial(
        pl.kernel,
        out_type=jax.ShapeDtypeStruct((tot, D), jnp.float32),
        mesh=mesh,
        scratch_types=[
            pltpu.VMEM((GPC, G), jnp.int32),
            pltpu.VMEM((GPC, G), jnp.int32),
            pltpu.VMEM((C, D), jnp.float32),
            pltpu.VMEM((C, D), jnp.float32),
            pltpu.SemaphoreType.DMA,
            pltpu.SemaphoreType.DMA,
        ],
        compiler_params=pltpu.CompilerParams(use_tc_tiling_on_sc=False),
    )
    def k(idx_hbm, table_hbm, out_hbm, idx_v0, idx_v1, rows_v0, rows_v1,
          gsem, wsem):
        idx_v = (idx_v0, idx_v1)
        rows_v = (rows_v0, rows_v1)
        wid = lax.axis_index("s") * NC + lax.axis_index("c")
        base_row = wid * rows_per_w

        def fire_gathers(ci, b):
            row0 = base_row + ci * GPC
            pltpu.sync_copy(idx_hbm.at[pl.ds(row0, GPC)], idx_v[b])
            for j in range(GPC):
                pltpu.async_copy(
                    table_hbm.at[idx_v[b].at[j]],
                    rows_v[b].at[pl.ds(j * G, G)],
                    gsem,
                )

        def wait_gathers(b):
            for j in range(GPC):
                pltpu.make_async_copy(
                    table_hbm.at[idx_v[b].at[j]],
                    rows_v[b].at[pl.ds(j * G, G)],
                    gsem,
                ).wait()

        def fire_write(ci, b):
            row0 = base_row + ci * GPC
            pltpu.async_copy(rows_v[b], out_hbm.at[pl.ds(row0 * G, C)], wsem)

        def wait_write(ci, b):
            row0 = base_row + ci * GPC
            pltpu.make_async_copy(
                rows_v[b], out_hbm.at[pl.ds(row0 * G, C)], wsem
            ).wait()

        fire_gathers(0, 0)

        @pl.loop(0, n_chunks, step=2)
        def chunk_loop(ci0):
            for b in range(2):
                ci = ci0 + b
                nb = 1 - b

                @pl.when(ci >= 1)
                def _():
                    wait_write(ci - 1, nb)

                @pl.when(ci + 1 < n_chunks)
                def _():
                    fire_gathers(ci + 1, nb)

                wait_gathers(b)
                fire_write(ci, b)

        wait_write(n_chunks - 1, (n_chunks - 1) % 2)

    return k


def _relayout_body(x_ref, o_ref):
    x = x_ref[...]
    x3 = x.reshape(BB, S // 4, 128)
    y = jnp.swapaxes(x3, 0, 1)
    o_ref[...] = jnp.swapaxes(y, 1, 2)


def _relayout(lin128, nb):
    nblk = nb // BB
    return pl.pallas_call(
        _relayout_body,
        grid=(nblk,),
        in_specs=[pl.BlockSpec((PPB, 128), lambda i: (i, 0))],
        out_specs=pl.BlockSpec((S // 4, 128, BB), lambda i: (0, 0, i)),
        out_shape=jax.ShapeDtypeStruct((S // 4, 128, nb), jnp.float32),
    )(lin128)


@functools.partial(jax.jit, static_argnums=(2,))
def _emb_lookup(src, table, nb):
    tot = nb * S
    idx2d = src.reshape(tot // G, G).astype(jnp.int32)
    lin = _gather_kernel(tot)(idx2d, table)
    out = _relayout(lin.reshape(tot // 4, 128), nb)
    return out.reshape(S, D, nb).transpose(2, 0, 1)


def kernel(src, table):
    nb, s = src.shape
    return _emb_lookup(src, table, nb)


# speedup vs baseline: 10.7772x; 1.0264x over previous
import functools

import jax
import jax.numpy as jnp
from jax import lax
from jax.experimental import pallas as pl
from jax.experimental.pallas import tpu as pltpu
from jax.experimental.pallas import tpu_sc as plsc

NC = 2
NS = 16
NW = NC * NS
D = 32
S = 200
G = 128
GPC = 8
C = G * GPC

BB = 256
PPB = BB * S // 4


def _gather_kernel(tot):
    per_w = tot // NW
    n_chunks = per_w // C
    rows_per_w = per_w // G
    assert n_chunks % 2 == 0

    mesh = plsc.VectorSubcoreMesh(core_axis_name="c", subcore_axis_name="s")

    @functools.partial(
        pl.kernel,
        out_type=jax.ShapeDtypeStruct((tot, D), jnp.float32),
        mesh=mesh,
        scratch_types=[
            pltpu.VMEM((C,), jnp.int32),
            pltpu.VMEM((C,), jnp.int32),
            pltpu.VMEM((C, D), jnp.float32),
            pltpu.VMEM((C, D), jnp.float32),
            pltpu.SemaphoreType.DMA,
            pltpu.SemaphoreType.DMA,
        ],
        compiler_params=pltpu.CompilerParams(use_tc_tiling_on_sc=False),
    )
    def k(idx_hbm, table_hbm, out_hbm, idx_v0, idx_v1, rows_v0, rows_v1,
          gsem, wsem):
        idx_v = (idx_v0, idx_v1)
        rows_v = (rows_v0, rows_v1)
        wid = lax.axis_index("s") * NC + lax.axis_index("c")
        base_row = wid * rows_per_w

        def fire_gathers(ci, b):
            i0 = (base_row + ci * GPC) * G
            pltpu.sync_copy(idx_hbm.at[pl.ds(i0, C)], idx_v[b])
            for j in range(GPC):
                pltpu.async_copy(
                    table_hbm.at[idx_v[b].at[pl.ds(j * G, G)]],
                    rows_v[b].at[pl.ds(j * G, G)],
                    gsem,
                )

        def wait_gathers(b):
            for j in range(GPC):
                pltpu.make_async_copy(
                    table_hbm.at[idx_v[b].at[pl.ds(j * G, G)]],
                    rows_v[b].at[pl.ds(j * G, G)],
                    gsem,
                ).wait()

        def fire_write(ci, b):
            row0 = base_row + ci * GPC
            pltpu.async_copy(rows_v[b], out_hbm.at[pl.ds(row0 * G, C)], wsem)

        def wait_write(ci, b):
            row0 = base_row + ci * GPC
            pltpu.make_async_copy(
                rows_v[b], out_hbm.at[pl.ds(row0 * G, C)], wsem
            ).wait()

        fire_gathers(0, 0)

        @pl.loop(0, n_chunks, step=2)
        def chunk_loop(ci0):
            for b in range(2):
                ci = ci0 + b
                nb = 1 - b

                @pl.when(ci >= 1)
                def _():
                    wait_write(ci - 1, nb)

                @pl.when(ci + 1 < n_chunks)
                def _():
                    fire_gathers(ci + 1, nb)

                wait_gathers(b)
                fire_write(ci, b)

        wait_write(n_chunks - 1, (n_chunks - 1) % 2)

    return k


def _relayout_body(x_ref, o_ref):
    x = x_ref[...]
    x3 = x.reshape(BB, S // 4, 128)
    y = jnp.swapaxes(x3, 0, 1)
    o_ref[...] = jnp.swapaxes(y, 1, 2)


def _relayout(lin128, nb):
    nblk = nb // BB
    return pl.pallas_call(
        _relayout_body,
        grid=(nblk,),
        in_specs=[pl.BlockSpec((PPB, 128), lambda i: (i, 0))],
        out_specs=pl.BlockSpec((S // 4, 128, BB), lambda i: (0, 0, i)),
        out_shape=jax.ShapeDtypeStruct((S // 4, 128, nb), jnp.float32),
    )(lin128)


@functools.partial(jax.jit, static_argnums=(2,))
def _emb_lookup(src, table, nb):
    tot = nb * S
    idx1d = src.reshape(tot).astype(jnp.int32)
    lin = _gather_kernel(tot)(idx1d, table)
    out = _relayout(lin.reshape(tot // 4, 128), nb)
    return out.reshape(S, D, nb).transpose(2, 0, 1)


def kernel(src, table):
    nb, s = src.shape
    return _emb_lookup(src, table, nb)
